# baseline (speedup 1.0000x reference)
import functools

import jax
import jax.numpy as jnp
from jax import lax
from jax.experimental import pallas as pl
from jax.experimental.pallas import tpu as pltpu
from jax.experimental.pallas import tpu_sc as plsc

VOCAB_N = 100000
EMBED_N = 100
TOKENS_N = 20
BATCH_N = 128
NUM_WORKERS = 32
ROWS_PER_W = BATCH_N // NUM_WORKERS
GATHER_N = ROWS_PER_W * TOKENS_N
CHUNKS = 7
W_PACKED = CHUNKS * 16


def _sc_body(inp_hbm, table_hbm, w_hbm, b_hbm, out_hbm,
             idx_v, rows_v, w_v, b_v, out_v, sem):
    nc = 2
    wid = lax.axis_index("s") * nc + lax.axis_index("c")

    pltpu.sync_copy(inp_hbm.at[wid], idx_v)
    copies = []
    for j0 in range(0, GATHER_N, 16):
        ivec = idx_v[pl.ds(j0, 16)]
        for l in range(16):
            copies.append(pltpu.async_copy(
                table_hbm.at[pl.ds(ivec[l], 1)],
                rows_v.at[pl.ds(j0 + l, 1)], sem))
    pltpu.sync_copy(w_hbm, w_v)
    pltpu.sync_copy(b_hbm, b_v)
    for c in copies:
        c.wait()

    zero = jnp.zeros((16,), jnp.float32)
    accs = [zero] * (2 * ROWS_PER_W)

    for t in range(TOKENS_N):
        for k in range(CHUNKS):
            xoff = 16 * k if k < 6 else EMBED_N - 16
            w0 = w_v[0, t, pl.ds(16 * k, 16)]
            w1 = w_v[1, t, pl.ds(16 * k, 16)]
            for r in range(ROWS_PER_W):
                x = rows_v[r * TOKENS_N + t, pl.ds(xoff, 16)]
                accs[2 * r] = accs[2 * r] + x * w0
                accs[2 * r + 1] = accs[2 * r + 1] + x * w1

    lane = lax.broadcasted_iota(jnp.int32, (16,), 0)
    out_vec = b_v[:]
    for j in range(2 * ROWS_PER_W):
        out_vec = jnp.where(lane == j, out_vec + jnp.sum(accs[j]), out_vec)
    out_v[:] = out_vec
    pltpu.sync_copy(out_v, out_hbm.at[wid])


_sc_kernel = pl.kernel(
    _sc_body,
    out_type=jax.ShapeDtypeStruct((NUM_WORKERS, 16), jnp.float32),
    mesh=plsc.VectorSubcoreMesh(core_axis_name="c", subcore_axis_name="s"),
    compiler_params=pltpu.CompilerParams(
        needs_layout_passes=False, use_tc_tiling_on_sc=False),
    scratch_types=[
        pltpu.VMEM((GATHER_N,), jnp.int32),
        pltpu.VMEM((GATHER_N, EMBED_N), jnp.float32),
        pltpu.VMEM((2, TOKENS_N, W_PACKED), jnp.float32),
        pltpu.VMEM((16,), jnp.float32),
        pltpu.VMEM((16,), jnp.float32),
        pltpu.SemaphoreType.DMA,
    ],
)


def kernel(input, table, W, b):
    inp_r = input.reshape(NUM_WORKERS, GATHER_N).astype(jnp.int32)
    wt = W.reshape(TOKENS_N, EMBED_N, 2).transpose(2, 0, 1)
    w_packed = jnp.concatenate(
        [wt[:, :, :84],
         jnp.zeros((2, TOKENS_N, 12), jnp.float32),
         wt[:, :, 84:]],
        axis=2)
    b16 = jnp.tile(b, 8)
    out = _sc_kernel(inp_r, table, w_packed, b16)
    return out[:, :2 * ROWS_PER_W].reshape(BATCH_N, 2)


# speedup vs baseline: 3.9756x; 3.9756x over previous
import jax
import jax.numpy as jnp
from jax import lax
from jax.experimental import pallas as pl
from jax.experimental.pallas import tpu as pltpu
from jax.experimental.pallas import tpu_sc as plsc

VOCAB_N = 100000
EMBED_N = 100
TOKENS_N = 20
BATCH_N = 128
NUM_WORKERS = 32
ROWS_PER_W = BATCH_N // NUM_WORKERS
GATHER_N = ROWS_PER_W * TOKENS_N
CHUNKS = 7
W_PACKED = 128


def _sc_body(inp_hbm, table_hbm, w_hbm, b_hbm, out_hbm,
             idx_v, rows_v, w_v, b_v, out_v, sem):
    nc = 2
    wid = lax.axis_index("s") * nc + lax.axis_index("c")

    pltpu.sync_copy(inp_hbm.at[pl.ds(wid, 1)], idx_v)
    copies = []
    for j0 in range(0, GATHER_N, 16):
        ivec = idx_v[0, pl.ds(j0, 16)]
        for l in range(16):
            copies.append(pltpu.async_copy(
                table_hbm.at[pl.ds(ivec[l], 1)],
                rows_v.at[pl.ds(j0 + l, 1)], sem))
    pltpu.sync_copy(w_hbm, w_v)
    pltpu.sync_copy(b_hbm, b_v)
    for c in copies:
        c.wait()

    zero = jnp.zeros((16,), jnp.float32)
    accs = [zero] * (2 * ROWS_PER_W)

    for t in range(TOKENS_N):
        for k in range(CHUNKS):
            xoff = 16 * k if k < 6 else EMBED_N - 16
            w0 = w_v[t, pl.ds(16 * k, 16)]
            w1 = w_v[TOKENS_N + t, pl.ds(16 * k, 16)]
            for r in range(ROWS_PER_W):
                x = rows_v[r * TOKENS_N + t, pl.ds(xoff, 16)]
                accs[2 * r] = accs[2 * r] + x * w0
                accs[2 * r + 1] = accs[2 * r + 1] + x * w1

    lane = lax.broadcasted_iota(jnp.int32, (16,), 0)
    out_vec = b_v[0, pl.ds(0, 16)]
    for j in range(2 * ROWS_PER_W):
        out_vec = jnp.where(lane == j, out_vec + jnp.sum(accs[j]), out_vec)
    out_v[0, pl.ds(0, 16)] = out_vec
    pltpu.sync_copy(out_v, out_hbm.at[pl.ds(wid, 1)])


_sc_kernel = pl.kernel(
    _sc_body,
    out_type=jax.ShapeDtypeStruct((NUM_WORKERS, 16), jnp.float32),
    mesh=plsc.VectorSubcoreMesh(core_axis_name="c", subcore_axis_name="s"),
    compiler_params=pltpu.CompilerParams(needs_layout_passes=False),
    scratch_types=[
        pltpu.VMEM((1, GATHER_N), jnp.int32),
        pltpu.VMEM((GATHER_N, EMBED_N), jnp.float32),
        pltpu.VMEM((2 * TOKENS_N, W_PACKED), jnp.float32),
        pltpu.VMEM((1, 16), jnp.float32),
        pltpu.VMEM((1, 16), jnp.float32),
        pltpu.SemaphoreType.DMA,
    ],
)


def kernel(input, table, W, b):
    inp_r = input.reshape(NUM_WORKERS, GATHER_N).astype(jnp.int32)
    wt = W.reshape(TOKENS_N, EMBED_N, 2).transpose(2, 0, 1)
    w_packed = jnp.concatenate(
        [wt[:, :, :84],
         jnp.zeros((2, TOKENS_N, 12), jnp.float32),
         wt[:, :, 84:],
         jnp.zeros((2, TOKENS_N, W_PACKED - 112), jnp.float32)],
        axis=2).reshape(2 * TOKENS_N, W_PACKED)
    b16 = jnp.tile(b, 8).reshape(1, 16)
    out = _sc_kernel(inp_r, table, w_packed, b16)
    return out[:, :2 * ROWS_PER_W].reshape(BATCH_N, 2)


# speedup vs baseline: 4.0124x; 1.0093x over previous
import jax
import jax.numpy as jnp
from jax import lax
from jax.experimental import pallas as pl
from jax.experimental.pallas import tpu as pltpu
from jax.experimental.pallas import tpu_sc as plsc

VOCAB_N = 100000
EMBED_N = 100
TOKENS_N = 20
BATCH_N = 128
NUM_WORKERS = 32
ROWS_PER_W = BATCH_N // NUM_WORKERS
GATHER_N = ROWS_PER_W * TOKENS_N
CHUNKS = 7
W_PACKED = 128


def _sc_body(inp_hbm, table_hbm, w_hbm, b_hbm, out_hbm,
             idx_v, rows_v, w_v, b_v, out_v, sem):
    nc = 2
    wid = lax.axis_index("s") * nc + lax.axis_index("c")

    pltpu.sync_copy(inp_hbm.at[pl.ds(wid, 1)], idx_v)
    copies = []
    for j0 in range(0, GATHER_N, 16):
        ivec = idx_v[0, pl.ds(j0, 16)]
        for l in range(16):
            copies.append(pltpu.async_copy(
                table_hbm.at[pl.ds(ivec[l], 1)],
                rows_v.at[pl.ds(j0 + l, 1)], sem))
    pltpu.sync_copy(w_hbm, w_v)
    pltpu.sync_copy(b_hbm, b_v)
    for c in copies:
        c.wait()

    zero = jnp.zeros((16,), jnp.float32)
    accs = [zero] * (2 * ROWS_PER_W)

    for t in range(TOKENS_N):
        for k in range(CHUNKS):
            xoff = 16 * k if k < 6 else EMBED_N - 16
            w0 = w_v[t, pl.ds(16 * k, 16)]
            w1 = w_v[TOKENS_N + t, pl.ds(16 * k, 16)]
            for r in range(ROWS_PER_W):
                x = rows_v[r * TOKENS_N + t, pl.ds(xoff, 16)]
                accs[2 * r] = accs[2 * r] + x * w0
                accs[2 * r + 1] = accs[2 * r + 1] + x * w1

    lane = lax.broadcasted_iota(jnp.int32, (16,), 0)
    out_vec = b_v[0, pl.ds(0, 16)]
    for j in range(2 * ROWS_PER_W):
        out_vec = jnp.where(lane == j, out_vec + jnp.sum(accs[j]), out_vec)
    out_v[0, pl.ds(0, 16)] = out_vec
    pltpu.sync_copy(out_v, out_hbm.at[pl.ds(wid, 1)])


_sc_kernel = pl.kernel(
    _sc_body,
    out_type=jax.ShapeDtypeStruct((NUM_WORKERS, 16), jnp.float32),
    mesh=plsc.VectorSubcoreMesh(core_axis_name="c", subcore_axis_name="s"),
    compiler_params=pltpu.CompilerParams(needs_layout_passes=False),
    scratch_types=[
        pltpu.VMEM((1, GATHER_N), jnp.int32),
        pltpu.VMEM((GATHER_N, EMBED_N), jnp.float32),
        pltpu.VMEM((2 * TOKENS_N, W_PACKED), jnp.float32),
        pltpu.VMEM((1, 16), jnp.float32),
        pltpu.VMEM((1, 16), jnp.float32),
        pltpu.SemaphoreType.DMA,
    ],
)


def _impl(input, table, W, b):
    inp_r = input.reshape(NUM_WORKERS, GATHER_N).astype(jnp.int32)
    wt = W.reshape(TOKENS_N, EMBED_N, 2).transpose(2, 0, 1)
    w_packed = jnp.concatenate(
        [wt[:, :, :84],
         jnp.zeros((2, TOKENS_N, 12), jnp.float32),
         wt[:, :, 84:],
         jnp.zeros((2, TOKENS_N, W_PACKED - 112), jnp.float32)],
        axis=2).reshape(2 * TOKENS_N, W_PACKED)
    b16 = jnp.tile(b, 8).reshape(1, 16)
    out = _sc_kernel(inp_r, table, w_packed, b16)
    return out[:, :2 * ROWS_PER_W].reshape(BATCH_N, 2)


from jax.experimental import layout as _jex_layout

_impl_jit_cache = {}


def _get_impl_jit():
    dev = jax.devices()[0]
    fn = _impl_jit_cache.get(dev)
    if fn is None:
        sharding = jax.sharding.SingleDeviceSharding(dev)

        def row_major(n):
            return _jex_layout.Format(
                _jex_layout.Layout(major_to_minor=tuple(range(n))), sharding)

        fn = jax.jit(
            _impl,
            in_shardings=(row_major(2), row_major(2), row_major(2),
                          row_major(1)),
        )
        _impl_jit_cache[dev] = fn
    return fn


def kernel(input, table, W, b):
    return _get_impl_jit()(input, table, W, b)


# speedup vs baseline: 4.0189x; 1.0016x over previous
import jax
import jax.numpy as jnp
from jax import lax
from jax.experimental import pallas as pl
from jax.experimental.pallas import tpu as pltpu
from jax.experimental.pallas import tpu_sc as plsc

VOCAB_N = 100000
EMBED_N = 100
TOKENS_N = 20
BATCH_N = 128
NUM_WORKERS = 32
ROWS_PER_W = BATCH_N // NUM_WORKERS
GATHER_N = ROWS_PER_W * TOKENS_N
CHUNKS = 7
W_PACKED = 128


def _sc_body(inp_hbm, table_hbm, w_hbm, b_hbm, out_hbm,
             idx_v, rows_v, w_v, b_v, out_v, sem):
    nc = 2
    wid = lax.axis_index("s") * nc + lax.axis_index("c")

    pltpu.sync_copy(inp_hbm.at[pl.ds(wid, 1)], idx_v)
    copies = []
    for j0 in range(0, GATHER_N, 16):
        ivec = idx_v[0, pl.ds(j0, 16)]
        for l in range(16):
            copies.append(pltpu.async_copy(
                table_hbm.at[pl.ds(ivec[l], 1)],
                rows_v.at[pl.ds(j0 + l, 1)], sem))
    pltpu.sync_copy(w_hbm, w_v)
    pltpu.sync_copy(b_hbm, b_v)
    for c in copies:
        c.wait()

    zero = jnp.zeros((16,), jnp.float32)
    accs = [zero] * (2 * ROWS_PER_W)

    for t in range(TOKENS_N):
        for k in range(CHUNKS):
            xoff = 16 * k if k < 6 else EMBED_N - 16
            w0 = w_v[t, pl.ds(16 * k, 16)]
            w1 = w_v[TOKENS_N + t, pl.ds(16 * k, 16)]
            for r in range(ROWS_PER_W):
                x = rows_v[r * TOKENS_N + t, pl.ds(xoff, 16)]
                accs[2 * r] = accs[2 * r] + x * w0
                accs[2 * r + 1] = accs[2 * r + 1] + x * w1

    lane = lax.broadcasted_iota(jnp.int32, (16,), 0)
    out_vec = b_v[0, pl.ds(0, 16)]
    for j in range(2 * ROWS_PER_W):
        out_vec = jnp.where(lane == j, out_vec + jnp.sum(accs[j]), out_vec)
    out_v[0, pl.ds(0, 16)] = out_vec
    pltpu.sync_copy(out_v, out_hbm.at[pl.ds(wid, 1)])


_sc_kernel = pl.kernel(
    _sc_body,
    out_type=jax.ShapeDtypeStruct((NUM_WORKERS, 16), jnp.float32),
    mesh=plsc.VectorSubcoreMesh(core_axis_name="c", subcore_axis_name="s"),
    compiler_params=pltpu.CompilerParams(needs_layout_passes=False),
    scratch_types=[
        pltpu.VMEM((1, GATHER_N), jnp.int32),
        pltpu.VMEM((GATHER_N, EMBED_N), jnp.float32),
        pltpu.VMEM((2 * TOKENS_N, W_PACKED), jnp.float32),
        pltpu.VMEM((1, 16), jnp.float32),
        pltpu.VMEM((1, 16), jnp.float32),
        pltpu.SemaphoreType.DMA,
    ],
)


def kernel(input, table, W, b):
    inp_r = input.reshape(NUM_WORKERS, GATHER_N).astype(jnp.int32)
    wt = W.reshape(TOKENS_N, EMBED_N, 2).transpose(2, 0, 1)
    w_packed = jnp.concatenate(
        [wt[:, :, :84],
         jnp.zeros((2, TOKENS_N, 12), jnp.float32),
         wt[:, :, 84:],
         jnp.zeros((2, TOKENS_N, W_PACKED - 112), jnp.float32)],
        axis=2).reshape(2 * TOKENS_N, W_PACKED)
    b16 = jnp.tile(b, 8).reshape(1, 16)
    out = _sc_kernel(inp_r, table, w_packed, b16)
    return out[:, :2 * ROWS_PER_W].reshape(BATCH_N, 2)
